# Initial kernel scaffold; baseline (speedup 1.0000x reference)
import jax
import jax.numpy as jnp
from jax.experimental import pallas as pl


def kernel(input, table, W_proj):
    raise NotImplementedError("write your pallas kernel here")


# speedup vs baseline: 1.7353x; 1.7353x over previous
import jax
import jax.numpy as jnp
from jax import lax
from jax.experimental import pallas as pl
from jax.experimental.pallas import tpu as pltpu
from jax.experimental.pallas import tpu_sc as plsc

BATCH = 16384
BAG_LEN = 20
DIM = 32
BASE_DIM = 128

NC = 2
NS = 16
NW = NC * NS
BAGS_PW = BATCH // NW
ROWS_PW = BAGS_PW * BAG_LEN
IDX_MINOR = 128
IDX_MAJOR = ROWS_PW // IDX_MINOR

CHUNK_BAGS = 32
CHUNK_ROWS = CHUNK_BAGS * BAG_LEN
GATHERS_PER_CHUNK = CHUNK_ROWS // IDX_MINOR
N_CHUNKS = BAGS_PW // CHUNK_BAGS


def _sc_pool_kernel(inp_hbm, table_hbm, pooled_hbm, idx_v, rows_a, rows_b,
                    pooled_v, sem_a, sem_b):
    wid = lax.axis_index("s") * NC + lax.axis_index("c")
    pltpu.sync_copy(inp_hbm.at[wid], idx_v)

    bufs = (rows_a, rows_b)
    sems = (sem_a, sem_b)
    descs = [None] * N_CHUNKS

    def start(g):
        buf = bufs[g % 2]
        sem = sems[g % 2]
        descs[g] = [
            pltpu.async_copy(
                table_hbm.at[idx_v.at[g * GATHERS_PER_CHUNK + k]],
                buf.at[pl.ds(k * IDX_MINOR, IDX_MINOR)],
                sem,
            )
            for k in range(GATHERS_PER_CHUNK)
        ]

    def pool(g):
        buf = bufs[g % 2]

        def bag_body(b, _):
            r0 = b * BAG_LEN
            acc0 = buf[r0, pl.ds(0, 16)]
            acc1 = buf[r0, pl.ds(16, 16)]
            for l in range(1, BAG_LEN):
                acc0 = acc0 + buf[r0 + l, pl.ds(0, 16)]
                acc1 = acc1 + buf[r0 + l, pl.ds(16, 16)]
            ob = g * CHUNK_BAGS + b
            pooled_v[ob, pl.ds(0, 16)] = acc0
            pooled_v[ob, pl.ds(16, 16)] = acc1
            return 0

        lax.fori_loop(0, CHUNK_BAGS, bag_body, 0)

    start(0)
    for g in range(N_CHUNKS):
        if g + 1 < N_CHUNKS:
            start(g + 1)
        for d in descs[g]:
            d.wait()
        pool(g)

    pltpu.sync_copy(pooled_v, pooled_hbm.at[wid])


def _sc_pool(inp3, table):
    mesh = plsc.VectorSubcoreMesh(
        core_axis_name="c", subcore_axis_name="s", num_cores=NC, num_subcores=NS
    )
    return pl.kernel(
        _sc_pool_kernel,
        out_type=jax.ShapeDtypeStruct((NW, BAGS_PW, DIM), jnp.float32),
        mesh=mesh,
        scratch_types=[
            pltpu.VMEM((IDX_MAJOR, IDX_MINOR), jnp.int32),
            pltpu.VMEM((CHUNK_ROWS, DIM), jnp.float32),
            pltpu.VMEM((CHUNK_ROWS, DIM), jnp.float32),
            pltpu.VMEM((BAGS_PW, DIM), jnp.float32),
            pltpu.SemaphoreType.DMA,
            pltpu.SemaphoreType.DMA,
        ],
        compiler_params=pltpu.CompilerParams(use_tc_tiling_on_sc=False),
    )(inp3, table)


def _proj_body(x_ref, w_ref, o_ref):
    o_ref[...] = lax.dot_general(
        x_ref[...], w_ref[...], (((1,), (1,)), ((), ())),
        preferred_element_type=jnp.float32,
    )


def _tc_proj(pooled, w_proj):
    blk = 2048
    grid = (BATCH // blk,)
    return pl.pallas_call(
        _proj_body,
        grid=grid,
        in_specs=[
            pl.BlockSpec((blk, DIM), lambda i: (i, 0)),
            pl.BlockSpec((BASE_DIM, DIM), lambda i: (0, 0)),
        ],
        out_specs=pl.BlockSpec((blk, BASE_DIM), lambda i: (i, 0)),
        out_shape=jax.ShapeDtypeStruct((BATCH, BASE_DIM), jnp.float32),
    )(pooled, w_proj)


def kernel(input, table, W_proj):
    inp3 = input.reshape(NW, IDX_MAJOR, IDX_MINOR)
    pooled = _sc_pool(inp3, table)
    return _tc_proj(pooled.reshape(BATCH, DIM), W_proj)
